# baseline (speedup 1.0000x reference)
import jax
import jax.numpy as jnp
from jax import lax
from jax.experimental import pallas as pl
from jax.experimental.pallas import tpu as pltpu
from jax.experimental.pallas import tpu_sc as plsc

ROWS = 16384
COLS = 1000
M = 200
MPAD = 208
L = 16

TC_ROWS = 10240
SC_ROWS = ROWS - TC_ROWS


BR = 1024
NCH = TC_ROWS // BR
KIN = 4
KOUT = 1
RIN = BR // KIN
ROUT = BR // KOUT


def _tc_body(mask_ref, x_hbm, o_hbm, w_ref, xv0, xv1, ov0, ov1, sin, sout):
    colid = lax.broadcasted_iota(jnp.int32, (COLS, MPAD), 0)
    mrow = jnp.broadcast_to(mask_ref[...], (COLS, MPAD))
    w_ref[...] = (colid == mrow).astype(jnp.bfloat16)

    xvs = (xv0, xv1)
    ovs = (ov0, ov1)

    def start_in(g):
        b = g % 2
        hs = []
        for k in range(KIN):
            h = pltpu.make_async_copy(
                x_hbm.at[pl.ds(g * BR + k * RIN, RIN)],
                xvs[b].at[pl.ds(k * RIN, RIN)],
                sin.at[b, k])
            h.start()
            hs.append(h)
        return hs

    def start_out(g):
        b = g % 2
        hs = []
        for k in range(KOUT):
            h = pltpu.make_async_copy(
                ovs[b].at[pl.ds(k * ROUT, ROUT)],
                o_hbm.at[pl.ds(g * BR + k * ROUT, ROUT)],
                sout.at[b, k])
            h.start()
            hs.append(h)
        return hs

    in_h = [None] * NCH
    out_h = [None] * NCH

    in_h[0] = start_in(0)

    for g in range(NCH):
        b = g % 2
        if g + 1 < NCH:
            in_h[g + 1] = start_in(g + 1)
        for h in in_h[g]:
            h.wait()
        if g >= 2:
            for h in out_h[g - 2]:
                h.wait()

        xb = xvs[b][...].astype(jnp.bfloat16)
        res = lax.dot_general(xb, w_ref[...], (((1,), (0,)), ((), ())),
                              preferred_element_type=jnp.float32)
        ovs[b][...] = res[:, :M]

        out_h[g] = start_out(g)

    for h in out_h[NCH - 2]:
        h.wait()
    for h in out_h[NCH - 1]:
        h.wait()


def _tc_part(x, mask2):
    return pl.pallas_call(
        _tc_body,
        in_specs=[
            pl.BlockSpec((1, MPAD), memory_space=pltpu.VMEM),
            pl.BlockSpec(memory_space=pl.ANY),
        ],
        out_specs=pl.BlockSpec(memory_space=pl.ANY),
        out_shape=jax.ShapeDtypeStruct((TC_ROWS, M), jnp.float32),
        scratch_shapes=[
            pltpu.VMEM((COLS, MPAD), jnp.bfloat16),
            pltpu.VMEM((BR, COLS), jnp.float32),
            pltpu.VMEM((BR, COLS), jnp.float32),
            pltpu.VMEM((BR, M), jnp.float32),
            pltpu.VMEM((BR, M), jnp.float32),
            pltpu.SemaphoreType.DMA((2, KIN)),
            pltpu.SemaphoreType.DMA((2, KOUT)),
        ],
    )(mask2, x)


NC = 2
NS = 16
NW = NC * NS
NMV = MPAD // L
RPW = SC_ROWS // NW
R = 32
NCHUNK = RPW // R


def _sc_body(x_hbm, mask_hbm, out_hbm,
             mask_v, xv0, xv1, ov0, ov1, si0, si1, so0, so1):
    wid = lax.axis_index("s") * NC + lax.axis_index("c")
    obase = wid * RPW
    base = TC_ROWS + obase

    pltpu.sync_copy(mask_hbm, mask_v)

    xvs = (xv0, xv1)
    ovs = (ov0, ov1)
    sis = (si0, si1)
    sos = (so0, so1)

    def start_in(g):
        b = g % 2
        h = pltpu.make_async_copy(
            x_hbm.at[pl.ds(base + g * R, R)], xvs[b], sis[b])
        h.start()
        return h

    def start_out(g):
        b = g % 2
        h = pltpu.make_async_copy(
            ovs[b], out_hbm.at[pl.ds(obase + g * R, R)], sos[b])
        h.start()
        return h

    in_h = [None] * NCHUNK
    out_h = [None] * NCHUNK

    in_h[0] = start_in(0)

    for g in range(NCHUNK):
        b = g % 2
        if g + 1 < NCHUNK:
            in_h[g + 1] = start_in(g + 1)
        in_h[g].wait()
        if g >= 2:
            out_h[g - 2].wait()

        xv, ov = xvs[b], ovs[b]

        def row(r, carry):
            rsplat = jnp.full((L,), 0, jnp.int32) + r
            for m in range(NMV):
                idx = mask_v[pl.ds(m * L, L)]
                vals = plsc.load_gather(xv, [rsplat, idx])
                if (m + 1) * L <= M:
                    ov[r, pl.ds(m * L, L)] = vals
                else:
                    cidx = m * L + lax.iota(jnp.int32, L)
                    plsc.store_scatter(ov, [rsplat, cidx], vals,
                                       mask=cidx < M)
            return carry

        lax.fori_loop(0, R, row, 0)

        out_h[g] = start_out(g)

    out_h[NCHUNK - 2].wait()
    out_h[NCHUNK - 1].wait()


def _sc_part(x, mask_padded):
    f = pl.kernel(
        _sc_body,
        out_type=jax.ShapeDtypeStruct((SC_ROWS, M), jnp.float32),
        mesh=plsc.VectorSubcoreMesh(core_axis_name="c", subcore_axis_name="s"),
        compiler_params=pltpu.CompilerParams(needs_layout_passes=False),
        scratch_types=[
            pltpu.VMEM((MPAD,), jnp.int32),
            pltpu.VMEM((R, COLS), jnp.float32),
            pltpu.VMEM((R, COLS), jnp.float32),
            pltpu.VMEM((R, M), jnp.float32),
            pltpu.VMEM((R, M), jnp.float32),
            pltpu.SemaphoreType.DMA,
            pltpu.SemaphoreType.DMA,
            pltpu.SemaphoreType.DMA,
            pltpu.SemaphoreType.DMA,
        ],
    )
    return f(x, mask_padded)


def kernel(x, mask):
    mask_padded = jnp.concatenate(
        [mask, jnp.zeros((MPAD - M,), jnp.int32)])
    mask2 = mask_padded.reshape(1, MPAD)
    out_sc = _sc_part(x, mask_padded)
    out_tc = _tc_part(x, mask2)
    return jnp.concatenate([out_tc, out_sc], axis=0)


# speedup vs baseline: 1.0773x; 1.0773x over previous
import jax
import jax.numpy as jnp
from jax import lax
from jax.experimental import pallas as pl
from jax.experimental.pallas import tpu as pltpu
from jax.experimental.pallas import tpu_sc as plsc

ROWS = 16384
COLS = 1000
M = 200
MPAD = 208
L = 16

TC_ROWS = 12288
SC_ROWS = ROWS - TC_ROWS


BR = 1024
NCH = TC_ROWS // BR
KIN = 4
KOUT = 1
RIN = BR // KIN
ROUT = BR // KOUT


def _tc_body(mask_ref, x_hbm, o_hbm, w_ref, xv0, xv1, ov0, ov1, sin, sout):
    colid = lax.broadcasted_iota(jnp.int32, (COLS, MPAD), 0)
    mrow = jnp.broadcast_to(mask_ref[...], (COLS, MPAD))
    w_ref[...] = (colid == mrow).astype(jnp.bfloat16)

    xvs = (xv0, xv1)
    ovs = (ov0, ov1)

    def start_in(g):
        b = g % 2
        hs = []
        for k in range(KIN):
            h = pltpu.make_async_copy(
                x_hbm.at[pl.ds(g * BR + k * RIN, RIN)],
                xvs[b].at[pl.ds(k * RIN, RIN)],
                sin.at[b, k])
            h.start()
            hs.append(h)
        return hs

    def start_out(g):
        b = g % 2
        hs = []
        for k in range(KOUT):
            h = pltpu.make_async_copy(
                ovs[b].at[pl.ds(k * ROUT, ROUT)],
                o_hbm.at[pl.ds(g * BR + k * ROUT, ROUT)],
                sout.at[b, k])
            h.start()
            hs.append(h)
        return hs

    in_h = [None] * NCH
    out_h = [None] * NCH

    in_h[0] = start_in(0)

    for g in range(NCH):
        b = g % 2
        if g + 1 < NCH:
            in_h[g + 1] = start_in(g + 1)
        for h in in_h[g]:
            h.wait()
        if g >= 2:
            for h in out_h[g - 2]:
                h.wait()

        xb = xvs[b][...].astype(jnp.bfloat16)
        res = lax.dot_general(xb, w_ref[...], (((1,), (0,)), ((), ())),
                              preferred_element_type=jnp.float32)
        ovs[b][...] = res[:, :M]

        out_h[g] = start_out(g)

    for h in out_h[NCH - 2]:
        h.wait()
    for h in out_h[NCH - 1]:
        h.wait()


def _tc_part(x, mask2):
    return pl.pallas_call(
        _tc_body,
        in_specs=[
            pl.BlockSpec((1, MPAD), memory_space=pltpu.VMEM),
            pl.BlockSpec(memory_space=pl.ANY),
        ],
        out_specs=pl.BlockSpec(memory_space=pl.ANY),
        out_shape=jax.ShapeDtypeStruct((TC_ROWS, M), jnp.float32),
        scratch_shapes=[
            pltpu.VMEM((COLS, MPAD), jnp.bfloat16),
            pltpu.VMEM((BR, COLS), jnp.float32),
            pltpu.VMEM((BR, COLS), jnp.float32),
            pltpu.VMEM((BR, M), jnp.float32),
            pltpu.VMEM((BR, M), jnp.float32),
            pltpu.SemaphoreType.DMA((2, KIN)),
            pltpu.SemaphoreType.DMA((2, KOUT)),
        ],
    )(mask2, x)


NC = 2
NS = 16
NW = NC * NS
NMV = MPAD // L
RPW = SC_ROWS // NW
R = 32
NCHUNK = RPW // R


def _sc_body(x_hbm, mask_hbm, out_hbm,
             mask_v, xv0, xv1, ov0, ov1, si0, si1, so0, so1):
    wid = lax.axis_index("s") * NC + lax.axis_index("c")
    obase = wid * RPW
    base = TC_ROWS + obase

    pltpu.sync_copy(mask_hbm, mask_v)

    xvs = (xv0, xv1)
    ovs = (ov0, ov1)
    sis = (si0, si1)
    sos = (so0, so1)

    def start_in(g):
        b = g % 2
        h = pltpu.make_async_copy(
            x_hbm.at[pl.ds(base + g * R, R)], xvs[b], sis[b])
        h.start()
        return h

    def start_out(g):
        b = g % 2
        h = pltpu.make_async_copy(
            ovs[b], out_hbm.at[pl.ds(obase + g * R, R)], sos[b])
        h.start()
        return h

    in_h = [None] * NCHUNK
    out_h = [None] * NCHUNK

    in_h[0] = start_in(0)

    for g in range(NCHUNK):
        b = g % 2
        if g + 1 < NCHUNK:
            in_h[g + 1] = start_in(g + 1)
        in_h[g].wait()
        if g >= 2:
            out_h[g - 2].wait()

        xv, ov = xvs[b], ovs[b]

        def row(r, carry):
            rsplat = jnp.full((L,), 0, jnp.int32) + r
            for m in range(NMV):
                idx = mask_v[pl.ds(m * L, L)]
                vals = plsc.load_gather(xv, [rsplat, idx])
                if (m + 1) * L <= M:
                    ov[r, pl.ds(m * L, L)] = vals
                else:
                    cidx = m * L + lax.iota(jnp.int32, L)
                    plsc.store_scatter(ov, [rsplat, cidx], vals,
                                       mask=cidx < M)
            return carry

        lax.fori_loop(0, R, row, 0)

        out_h[g] = start_out(g)

    out_h[NCHUNK - 2].wait()
    out_h[NCHUNK - 1].wait()


def _sc_part(x, mask_padded):
    f = pl.kernel(
        _sc_body,
        out_type=jax.ShapeDtypeStruct((SC_ROWS, M), jnp.float32),
        mesh=plsc.VectorSubcoreMesh(core_axis_name="c", subcore_axis_name="s"),
        compiler_params=pltpu.CompilerParams(needs_layout_passes=False),
        scratch_types=[
            pltpu.VMEM((MPAD,), jnp.int32),
            pltpu.VMEM((R, COLS), jnp.float32),
            pltpu.VMEM((R, COLS), jnp.float32),
            pltpu.VMEM((R, M), jnp.float32),
            pltpu.VMEM((R, M), jnp.float32),
            pltpu.SemaphoreType.DMA,
            pltpu.SemaphoreType.DMA,
            pltpu.SemaphoreType.DMA,
            pltpu.SemaphoreType.DMA,
        ],
    )
    return f(x, mask_padded)


def kernel(x, mask):
    mask_padded = jnp.concatenate(
        [mask, jnp.zeros((MPAD - M,), jnp.int32)])
    mask2 = mask_padded.reshape(1, MPAD)
    out_sc = _sc_part(x, mask_padded)
    out_tc = _tc_part(x, mask2)
    return jnp.concatenate([out_tc, out_sc], axis=0)
